# baseline (speedup 1.0000x reference)
import functools

import jax
import jax.numpy as jnp
from jax import lax
from jax.experimental import pallas as pl
from jax.experimental.pallas import tpu as pltpu
from jax.experimental.pallas import tpu_sc as plsc

D_MODEL = 1024
N_IDS = 4 * 4096

_NC, _NS = 2, 16
_NW = _NC * _NS
_PER_W = N_IDS // _NW
_GROUPS = _PER_W // 16


def _embed_body(table_hbm, idx_hbm, out_hbm, idx_v, sem):
    wid = lax.axis_index("s") * _NC + lax.axis_index("c")
    base = wid * _PER_W
    pltpu.sync_copy(idx_hbm.at[pl.ds(base, _PER_W)], idx_v)

    def step(g, carry):
        vec = idx_v[pl.ds(g * 16, 16)]
        row0 = base + g * 16
        for j in range(16):
            pltpu.async_copy(table_hbm.at[vec[j]], out_hbm.at[row0 + j], sem)
        return carry

    lax.fori_loop(0, _GROUPS, step, 0)
    pltpu.make_async_copy(
        table_hbm.at[pl.ds(0, _PER_W)], out_hbm.at[pl.ds(base, _PER_W)], sem
    ).wait()


@jax.jit
def _embed_lookup(table, ids):
    mesh = plsc.VectorSubcoreMesh(core_axis_name="c", subcore_axis_name="s")
    run = pl.kernel(
        _embed_body,
        mesh=mesh,
        out_type=jax.ShapeDtypeStruct((N_IDS, D_MODEL), jnp.float32),
        scratch_types=[
            pltpu.VMEM((_PER_W,), jnp.int32),
            pltpu.SemaphoreType.DMA,
        ],
    )
    return run(table, ids)


def kernel(input_ids, is_node, node_features, edge_index, mapping, embed_weight):
    ids = input_ids.reshape(-1)
    out = _embed_lookup(embed_weight, ids)
    return out.reshape(input_ids.shape[0], input_ids.shape[1], D_MODEL)


# speedup vs baseline: 30.3827x; 30.3827x over previous
import functools

import jax
import jax.numpy as jnp
from jax import lax
from jax.experimental import pallas as pl
from jax.experimental.pallas import tpu as pltpu
from jax.experimental.pallas import tpu_sc as plsc

D_MODEL = 1024
N_SEQ = 4
L_SEQ = 4096

_NC, _NS = 2, 16
_NW = _NC * _NS
_PER_W = (N_SEQ * L_SEQ) // _NW
_W_PER_SEQ = L_SEQ // _PER_W
_CHUNK = 32
_NCHUNK = _PER_W // _CHUNK


def _embed_body(table_hbm, idx_hbm, out_hbm, idx_v, rows0, rows1,
                gsem0, gsem1, ssem0, ssem1):
    wid = lax.axis_index("s") * _NC + lax.axis_index("c")
    seq = wid // _W_PER_SEQ
    col = (wid % _W_PER_SEQ) * _PER_W
    pltpu.sync_copy(idx_hbm.at[seq, pl.ds(col, _PER_W)], idx_v)

    bufs = (rows0, rows1)
    gsems = (gsem0, gsem1)
    ssems = (ssem0, ssem1)

    def gather(c, b):
        return pltpu.async_copy(
            table_hbm.at[idx_v.at[pl.ds(c * _CHUNK, _CHUNK)]], bufs[b], gsems[b]
        )

    def scatter(c, b):
        return pltpu.async_copy(
            bufs[b], out_hbm.at[seq, pl.ds(col + c * _CHUNK, _CHUNK)], ssems[b]
        )

    gather(0, 0)
    gather(1, 1)

    def pair(i, carry):
        for b in (0, 1):
            c = 2 * i + b
            pltpu.make_async_copy(
                table_hbm.at[pl.ds(0, _CHUNK)], bufs[b], gsems[b]
            ).wait()
            scatter(c, b)
            pltpu.make_async_copy(
                bufs[b], out_hbm.at[seq, pl.ds(col, _CHUNK)], ssems[b]
            ).wait()

            @pl.when(c + 2 < _NCHUNK)
            def _():
                gather(c + 2, b)

        return carry

    lax.fori_loop(0, _NCHUNK // 2, pair, 0)


@jax.jit
def _embed_lookup(table, ids):
    mesh = plsc.VectorSubcoreMesh(core_axis_name="c", subcore_axis_name="s")
    run = pl.kernel(
        _embed_body,
        mesh=mesh,
        out_type=jax.ShapeDtypeStruct((N_SEQ, L_SEQ, D_MODEL), jnp.float32),
        scratch_types=[
            pltpu.VMEM((_PER_W,), jnp.int32),
            pltpu.VMEM((_CHUNK, D_MODEL), jnp.float32),
            pltpu.VMEM((_CHUNK, D_MODEL), jnp.float32),
            pltpu.SemaphoreType.DMA,
            pltpu.SemaphoreType.DMA,
            pltpu.SemaphoreType.DMA,
            pltpu.SemaphoreType.DMA,
        ],
    )
    return run(table, ids)


def kernel(input_ids, is_node, node_features, edge_index, mapping, embed_weight):
    return _embed_lookup(embed_weight, input_ids)
